# baseline (speedup 1.0000x reference)
import functools

import jax
import jax.numpy as jnp
import numpy as np
from jax import lax
from jax.experimental import pallas as pl
from jax.experimental.pallas import tpu as pltpu
from jax.experimental.pallas import tpu_sc as plsc

BATCH = 16
LANES = 16
NC = 2
NS = 16
NW = NC * NS
CHUNK = 128
NBUF = 4
RU = 8
TBLK = 8192

_LO = np.r_[0:16, 32:48]
_HI = np.r_[16:32, 48:64]


def _tc_repack(table_t):
    d, v = table_t.shape
    n_blk = (v + TBLK - 1) // TBLK
    p_lo = np.zeros((d, 32), np.float32)
    p_lo[_LO, np.arange(32)] = 1.0
    p_hi = np.zeros((d, 32), np.float32)
    p_hi[_HI, np.arange(32)] = 1.0

    def body(a_ref, pl_ref, ph_ref, o_ref):
        a = a_ref[...].astype(jnp.bfloat16)
        dims = (((0,), (0,)), ((), ()))
        lo = lax.dot_general(a, pl_ref[...], dims,
                             preferred_element_type=jnp.float32)
        hi = lax.dot_general(a, ph_ref[...], dims,
                             preferred_element_type=jnp.float32)
        lo_u = lax.bitcast_convert_type(
            lo.astype(jnp.bfloat16), jnp.uint16).astype(jnp.uint32)
        hi_u = lax.bitcast_convert_type(
            hi.astype(jnp.bfloat16), jnp.uint16).astype(jnp.uint32)
        o_ref[...] = lax.bitcast_convert_type(
            (hi_u << 16) | lo_u, jnp.float32)

    return pl.pallas_call(
        body,
        grid=(n_blk,),
        in_specs=[
            pl.BlockSpec((d, TBLK), lambda j: (0, j)),
            pl.BlockSpec((d, 32), lambda j: (0, 0)),
            pl.BlockSpec((d, 32), lambda j: (0, 0)),
        ],
        out_specs=pl.BlockSpec((TBLK, 32), lambda j: (j, 0)),
        out_shape=jax.ShapeDtypeStruct((n_blk * TBLK, 32), jnp.float32),
    )(table_t, jnp.asarray(p_lo, jnp.bfloat16), jnp.asarray(p_hi, jnp.bfloat16))


def _sc_partial_sums(rowidx3, table32, n_chunks):
    width = table32.shape[1]
    mesh = plsc.VectorSubcoreMesh(
        core_axis_name="c", subcore_axis_name="s",
        num_cores=NC, num_subcores=NS)

    @functools.partial(
        pl.kernel,
        out_type=jax.ShapeDtypeStruct((NC, NS, 64), jnp.float32),
        mesh=mesh,
        scratch_types=[
            pltpu.VMEM((n_chunks, CHUNK), jnp.int32),
            pltpu.VMEM((NBUF, CHUNK, width), jnp.float32),
            pltpu.VMEM((64,), jnp.float32),
            pltpu.SemaphoreType.DMA,
            pltpu.SemaphoreType.DMA,
            pltpu.SemaphoreType.DMA,
            pltpu.SemaphoreType.DMA,
        ],
        compiler_params=pltpu.CompilerParams(
            use_tc_tiling_on_sc=False, needs_layout_passes=False),
    )
    def body(rows_hbm, table_hbm, out_hbm, idx_v, rows_v, acc_v, *sems):
        ci = lax.axis_index("c")
        si = lax.axis_index("s")
        wid = si * NC + ci
        pltpu.sync_copy(rows_hbm.at[wid], idx_v)

        for p in range(NBUF):
            pltpu.async_copy(table_hbm.at[idx_v.at[p]], rows_v.at[p], sems[p])

        zero = jnp.zeros((LANES,), jnp.float32)
        himask = jnp.full((LANES,), 0xFFFF0000, jnp.uint32)
        shift = jnp.full((LANES,), 16, jnp.uint32)

        def accum_row(rows_p, row, accs4):
            out = []
            for k in range(2):
                words = rows_p[row, pl.ds(k * LANES, LANES)]
                u = plsc.bitcast(words, jnp.uint32)
                lo = plsc.bitcast(u << shift, jnp.float32)
                hi = plsc.bitcast(u & himask, jnp.float32)
                out.append(accs4[2 * k] + lo)
                out.append(accs4[2 * k + 1] + hi)
            return out

        def accum_chunk(rows_p, accs):
            def row_body(r, accs8):
                accs8 = list(accs8)
                for u in range(0, RU, 2):
                    row = r * RU + u
                    accs8[:4] = accum_row(rows_p, row, accs8[:4])
                    accs8[4:] = accum_row(rows_p, row + 1, accs8[4:])
                return tuple(accs8)

            accs8 = accs + (zero, zero, zero, zero)
            accs8 = lax.fori_loop(0, CHUNK // RU, row_body, accs8)
            return tuple(accs8[k] + accs8[4 + k] for k in range(4))

        def outer(t, accs):
            for p in range(NBUF):
                j = t * NBUF + p
                pltpu.make_async_copy(
                    table_hbm.at[idx_v.at[j]], rows_v.at[p], sems[p]).wait()
                accs = accum_chunk(rows_v.at[p], accs)

                @pl.when(j + NBUF < n_chunks)
                def _():
                    pltpu.async_copy(
                        table_hbm.at[idx_v.at[j + NBUF]], rows_v.at[p], sems[p])
            return accs

        accs = lax.fori_loop(0, n_chunks // NBUF, outer,
                             (zero, zero, zero, zero))
        for k in range(4):
            acc_v[pl.ds(k * LANES, LANES)] = accs[k]
        pltpu.sync_copy(acc_v, out_hbm.at[ci, si])

    return body(rowidx3, table32)


def _tc_head(partials, fcw_t, bias2d, inv_count):
    num_class = fcw_t.shape[1]

    def body(p_ref, w_ref, b_ref, o_ref):
        pooled = (p_ref[0] + p_ref[1]) * inv_count
        o_ref[...] = jnp.dot(pooled, w_ref[...],
                             preferred_element_type=jnp.float32) + b_ref[...]

    return pl.pallas_call(
        body,
        out_shape=jax.ShapeDtypeStruct((BATCH, num_class), jnp.float32),
    )(partials, fcw_t, bias2d)


def kernel(text, emb_weight, fc_weight, fc_bias):
    n = text.shape[0]
    count = n // BATCH
    per_w = n // NW
    n_chunks = per_w // CHUNK
    assert n % (NW * CHUNK) == 0 and count % per_w == 0
    assert n_chunks % NBUF == 0

    table32 = _tc_repack(emb_weight.T)
    rowidx3 = text.astype(jnp.int32).reshape(NW, n_chunks, CHUNK)

    partials = _sc_partial_sums(rowidx3, table32, n_chunks)
    fcw_t = fc_weight.T
    bias2d = fc_bias.reshape(1, -1)
    return _tc_head(partials, fcw_t, bias2d, 1.0 / count)


# speedup vs baseline: 2.1097x; 2.1097x over previous
import functools

import jax
import jax.numpy as jnp
from jax import lax
from jax.experimental import pallas as pl
from jax.experimental.pallas import tpu as pltpu
from jax.experimental.pallas import tpu_sc as plsc

BATCH = 16
LANES = 16
NC = 2
NS = 16
NW = NC * NS
CHUNK = 128
NBUF = 4
RU = 8
TBLK = 8192


def _tc_repack(table_t):
    d, v = table_t.shape
    n_blk = (v + TBLK - 1) // TBLK
    half = TBLK // 2

    def body(a_ref, o_ref):
        ident = jnp.eye(d, dtype=jnp.float32)
        a = a_ref[...]
        t = a.T
        t0 = t[:half]
        t1 = t[half:]
        o_ref[:, :d] = t0
        o_ref[:, d:] = t1

    return pl.pallas_call(
        body,
        grid=(n_blk,),
        in_specs=[pl.BlockSpec((d, TBLK), lambda j: (0, j))],
        out_specs=pl.BlockSpec((half, 2 * d), lambda j: (j, 0)),
        out_shape=jax.ShapeDtypeStruct((n_blk * half, 2 * d), jnp.float32),
    )(table_t)


def _sc_partial_sums(rowidx3, table64, n_chunks):
    d = table64.shape[1]
    mesh = plsc.VectorSubcoreMesh(
        core_axis_name="c", subcore_axis_name="s",
        num_cores=NC, num_subcores=NS)

    @functools.partial(
        pl.kernel,
        out_type=jax.ShapeDtypeStruct((NC, NS, d), jnp.float32),
        mesh=mesh,
        scratch_types=[
            pltpu.VMEM((n_chunks, CHUNK), jnp.int32),
            pltpu.VMEM((NBUF, CHUNK, d), jnp.float32),
            pltpu.VMEM((d,), jnp.float32),
            pltpu.SemaphoreType.DMA,
            pltpu.SemaphoreType.DMA,
            pltpu.SemaphoreType.DMA,
            pltpu.SemaphoreType.DMA,
        ],
        compiler_params=pltpu.CompilerParams(use_tc_tiling_on_sc=False),
    )
    def body(rows_hbm, table_hbm, out_hbm, idx_v, rows_v, acc_v, *sems):
        ci = lax.axis_index("c")
        si = lax.axis_index("s")
        wid = si * NC + ci
        pltpu.sync_copy(rows_hbm.at[wid], idx_v)

        for p in range(NBUF):
            pltpu.async_copy(table_hbm.at[idx_v.at[p]], rows_v.at[p], sems[p])

        zero = jnp.zeros((LANES,), jnp.float32)

        def accum_chunk(rows_p, accs):
            def row_body(r, accs8):
                accs8 = list(accs8)
                for u in range(0, RU, 2):
                    row = r * RU + u
                    for k in range(4):
                        accs8[k] = accs8[k] + rows_p[
                            row, pl.ds(k * LANES, LANES)]
                        accs8[4 + k] = accs8[4 + k] + rows_p[
                            row + 1, pl.ds(k * LANES, LANES)]
                return tuple(accs8)

            accs8 = accs + (zero, zero, zero, zero)
            accs8 = lax.fori_loop(0, CHUNK // RU, row_body, accs8)
            return tuple(accs8[k] + accs8[4 + k] for k in range(4))

        def outer(t, accs):
            for p in range(NBUF):
                j = t * NBUF + p
                pltpu.make_async_copy(
                    table_hbm.at[idx_v.at[j]], rows_v.at[p], sems[p]).wait()
                accs = accum_chunk(rows_v.at[p], accs)

                @pl.when(j + NBUF < n_chunks)
                def _():
                    pltpu.async_copy(
                        table_hbm.at[idx_v.at[j + NBUF]], rows_v.at[p], sems[p])
            return accs

        accs = lax.fori_loop(0, n_chunks // NBUF, outer,
                             (zero, zero, zero, zero))
        for k in range(4):
            acc_v[pl.ds(k * LANES, LANES)] = accs[k]
        pltpu.sync_copy(acc_v, out_hbm.at[ci, si])

    return body(rowidx3, table64)


def _tc_head(partials, fcw_t, bias2d, inv_count):
    num_class = fcw_t.shape[1]

    def body(p_ref, w_ref, b_ref, o_ref):
        pooled = (p_ref[0] + p_ref[1]) * inv_count
        o_ref[...] = jnp.dot(pooled, w_ref[...],
                             preferred_element_type=jnp.float32) + b_ref[...]

    return pl.pallas_call(
        body,
        out_shape=jax.ShapeDtypeStruct((BATCH, num_class), jnp.float32),
    )(partials, fcw_t, bias2d)


def kernel(text, emb_weight, fc_weight, fc_bias):
    n = text.shape[0]
    count = n // BATCH
    per_w = n // NW
    n_chunks = per_w // CHUNK
    assert n % (NW * CHUNK) == 0 and count % per_w == 0
    assert n_chunks % NBUF == 0

    table2 = _tc_repack(emb_weight.T)
    table64 = table2.reshape(-1, table2.shape[1] // 2)

    half = TBLK // 2
    tok = text.astype(jnp.int32)
    row64 = (tok // TBLK) * TBLK + (tok % half) * 2 + (tok // half) % 2
    rowidx3 = row64.reshape(NW, n_chunks, CHUNK)

    partials = _sc_partial_sums(rowidx3, table64, n_chunks)
    fcw_t = fc_weight.T
    bias2d = fc_bias.reshape(1, -1)
    return _tc_head(partials, fcw_t, bias2d, 1.0 / count)
